# Initial kernel scaffold; baseline (speedup 1.0000x reference)
import jax
import jax.numpy as jnp
from jax.experimental import pallas as pl


def kernel(x, LUT):
    raise NotImplementedError("write your pallas kernel here")


# speedup vs baseline: 726.5898x; 726.5898x over previous
import functools

import jax
import jax.numpy as jnp
from jax import lax
from jax.experimental import pallas as pl
from jax.experimental.pallas import tpu as pltpu
from jax.experimental.pallas import tpu_sc as plsc

DIM = 33
D2 = DIM * DIM
D3 = DIM * DIM * DIM
D3P = D3 + 7
LUT_WORDS = 3 * D3P

NTILES = 32
LANES = 16
BROWS = 8
BCOLS = 256


def _make_sc_kernel(batch, h, w):
    rows_per_tile = batch * h // NTILES
    tiles_per_plane = h // rows_per_tile
    row_steps = rows_per_tile // BROWS
    col_steps = w // BCOLS
    nsteps = row_steps * col_steps

    mesh = plsc.VectorSubcoreMesh(core_axis_name="c", subcore_axis_name="s")

    @functools.partial(
        pl.kernel,
        mesh=mesh,
        compiler_params=pltpu.CompilerParams(needs_layout_passes=False),
        out_type=jax.ShapeDtypeStruct((batch, 3, h, w), jnp.float32),
        scratch_types=[
            pltpu.VMEM((LUT_WORDS,), jnp.float32),
            pltpu.VMEM((BROWS, BCOLS), jnp.float32),
            pltpu.VMEM((BROWS, BCOLS), jnp.float32),
            pltpu.VMEM((BROWS, BCOLS), jnp.float32),
            pltpu.VMEM((BROWS, BCOLS), jnp.float32),
            pltpu.VMEM((BROWS, BCOLS), jnp.float32),
            pltpu.VMEM((BROWS, BCOLS), jnp.float32),
        ],
    )
    def sc_kernel(x_hbm, lut_hbm, out_hbm, lut_v, rbuf, gbuf, bbuf, orb, ogb, obb):
        wid = lax.axis_index("s") * 2 + lax.axis_index("c")
        pltpu.sync_copy(lut_hbm, lut_v)
        bidx = wid // tiles_per_plane
        tile_row0 = (wid % tiles_per_plane) * rows_per_tile

        views = [lut_v.at[pl.ds(c * D3P, D3P)] for c in range(3)]
        corner_offs = (1, DIM, DIM + 1, D2, D2 + 1, D2 + DIM, D2 + DIM + 1)

        def step_body(ci, carry):
            r0 = tile_row0 + (ci // col_steps) * BROWS
            co = (ci % col_steps) * BCOLS
            src = lambda ch: x_hbm.at[bidx, ch, pl.ds(r0, BROWS), pl.ds(co, BCOLS)]
            dst = lambda ch: out_hbm.at[bidx, ch, pl.ds(r0, BROWS), pl.ds(co, BCOLS)]
            pltpu.sync_copy(src(0), rbuf)
            pltpu.sync_copy(src(1), gbuf)
            pltpu.sync_copy(src(2), bbuf)

            @plsc.parallel_loop(0, BCOLS, step=LANES, unroll=1)
            def grp(s):
                for j in range(BROWS):
                    rr = rbuf[j, pl.ds(s, LANES)]
                    gg = gbuf[j, pl.ds(s, LANES)]
                    bb = bbuf[j, pl.ds(s, LANES)]

                    def prep(v):
                        xs = jnp.minimum(jnp.maximum(v, 0.0), 1.0) * float(DIM - 1)
                        i0 = jnp.minimum(xs.astype(jnp.int32), DIM - 2)
                        f = xs - i0.astype(jnp.float32)
                        return i0, f

                    r0i, fr = prep(rr)
                    g0i, fg = prep(gg)
                    b0i, fb = prep(bb)
                    base = r0i * D2 + g0i * DIM + b0i
                    wb0 = 1.0 - fb
                    wg0 = 1.0 - fg
                    w00 = wg0 * wb0
                    w01 = wg0 * fb
                    w10 = fg * wb0
                    w11 = fg * fb
                    wr0 = 1.0 - fr

                    idxs = [base] + [base + co2 for co2 in corner_offs]
                    wlohi = (w00, w01, w10, w11)
                    for c, obuf in ((0, orb), (1, ogb), (2, obb)):
                        lo = plsc.load_gather(views[c], [idxs[0]]) * w00
                        for k in (1, 2, 3):
                            lo += plsc.load_gather(views[c], [idxs[k]]) * wlohi[k]
                        hi = plsc.load_gather(views[c], [idxs[4]]) * w00
                        for k in (5, 6, 7):
                            hi += plsc.load_gather(views[c], [idxs[k]]) * wlohi[k - 4]
                        obuf[j, pl.ds(s, LANES)] = lo * wr0 + hi * fr

            pltpu.sync_copy(orb, dst(0))
            pltpu.sync_copy(ogb, dst(1))
            pltpu.sync_copy(obb, dst(2))
            return carry

        lax.fori_loop(0, nsteps, step_body, 0)

    return sc_kernel


def kernel(x, LUT):
    batch, _, h, w = x.shape
    sc = _make_sc_kernel(batch, h, w)
    lut_flat = jnp.pad(LUT.reshape(3, D3), ((0, 0), (0, D3P - D3))).reshape(-1)
    return sc(x, lut_flat)
